# speedup vs baseline: 1.2792x; 1.2792x over previous
import functools

import jax
import jax.numpy as jnp
from jax.experimental import pallas as pl
from jax.experimental.pallas import tpu as pltpu


def _pick_tile(n, unit, cap):
    best = None
    t = unit
    limit = min(n, cap)
    while t <= limit:
        if n % t == 0:
            best = t
        t += unit
    return best if best is not None else n


def _fused_bn_fc_kernel(x_ref, g_ref, bt_ref, w_ref, bias_ref,
                        o_ref,
                        wb_ref, sum_ref, sumsq_ref, p_ref,
                        *, inv_n, eps, tb):
    j = pl.program_id(0)
    C = sum_ref.shape[0]
    HW = sum_ref.shape[1]

    @pl.when(j == 0)
    def _():
        sum_ref[...] = jnp.zeros_like(sum_ref)
        sumsq_ref[...] = jnp.zeros_like(sumsq_ref)
        wb_ref[...] = w_ref[...].astype(jnp.bfloat16)

    xf = x_ref[...]
    sum_ref[...] += jnp.sum(xf, axis=0)
    sumsq_ref[...] += jnp.sum(xf * xf, axis=0)

    xb = xf.astype(jnp.bfloat16)
    for c in range(C):
        pc = jax.lax.dot_general(
            xb[:, c, :], wb_ref[:, c, :],
            dimension_numbers=(((1,), (1,)), ((), ())),
            preferred_element_type=jnp.float32)
        p_ref[c, pl.ds(j * tb, tb), :] = pc

    @pl.when(j == pl.num_programs(0) - 1)
    def _():
        mean = jnp.sum(sum_ref[...], axis=1, keepdims=True) * inv_n
        var = jnp.sum(sumsq_ref[...], axis=1, keepdims=True) * inv_n - mean * mean
        var = jnp.maximum(var, 0.0)
        s = g_ref[...] * jax.lax.rsqrt(var + eps)
        t = bt_ref[...] - mean * s

        ones_row = jnp.ones((1, HW), dtype=jnp.bfloat16)
        cst = bias_ref[...]
        for c in range(C):
            wsum_c = jax.lax.dot_general(
                ones_row, wb_ref[:, c, :],
                dimension_numbers=(((1,), (1,)), ((), ())),
                preferred_element_type=jnp.float32)
            cst = cst + t[c:c + 1, :] * wsum_c

        acc = jnp.zeros(o_ref.shape, dtype=jnp.float32)
        for c in range(C):
            acc = acc + p_ref[c] * s[c:c + 1, :]
        o_ref[...] = acc + cst


def kernel(x, gamma, beta, weight, bias):
    B, C, H, W = x.shape
    HW = H * W
    K = weight.shape[0]

    x3 = x.reshape(B, C, HW)
    w3 = weight.reshape(K, C, HW)

    tb = _pick_tile(B, 8, max(8, min(256, B // 4)))
    grid = (B // tb,)

    out = pl.pallas_call(
        functools.partial(_fused_bn_fc_kernel,
                          inv_n=1.0 / float(B * HW), eps=1e-5, tb=tb),
        out_shape=jax.ShapeDtypeStruct((B, K), jnp.float32),
        grid=grid,
        in_specs=[pl.BlockSpec((tb, C, HW), lambda j: (j, 0, 0)),
                  pl.BlockSpec((C, 1), lambda j: (0, 0)),
                  pl.BlockSpec((C, 1), lambda j: (0, 0)),
                  pl.BlockSpec((K, C, HW), lambda j: (0, 0, 0)),
                  pl.BlockSpec((1, K), lambda j: (0, 0))],
        out_specs=pl.BlockSpec((B, K), lambda j: (0, 0)),
        scratch_shapes=[pltpu.VMEM((K, C, HW), jnp.bfloat16),
                        pltpu.VMEM((C, HW), jnp.float32),
                        pltpu.VMEM((C, HW), jnp.float32),
                        pltpu.VMEM((C, B, K), jnp.float32)],
        compiler_params=pltpu.CompilerParams(
            dimension_semantics=("arbitrary",),
            vmem_limit_bytes=56 * 1024 * 1024),
    )(x3, gamma.reshape(C, 1), beta.reshape(C, 1), w3, bias.reshape(1, K))
    return out
